# Initial kernel scaffold; baseline (speedup 1.0000x reference)
import jax
import jax.numpy as jnp
from jax.experimental import pallas as pl


def kernel(x, edge_index, gen_lin_src_w, gen_lin_src_b, gen_lin_dst_w, gen_lin_dst_b, gen_mlp_w1, gen_mlp_b1, gen_bn_gamma, gen_bn_beta, gen_mlp_w2, gen_mlp_b2, sage_lin_l_w, sage_lin_l_b, sage_lin_r_w, tr_q_w, tr_q_b, tr_k_w, tr_k_b, tr_v_w, tr_v_b, tr_skip_w, tr_skip_b):
    raise NotImplementedError("write your pallas kernel here")


# speedup vs baseline: 4.9890x; 4.9890x over previous
import functools

import jax
import jax.numpy as jnp
from jax import lax
from jax.experimental import pallas as pl
from jax.experimental.pallas import tpu as pltpu
from jax.experimental.pallas import tpu_sc as plsc

N = 10000
NPAD = 10240
E = 160000
H = 256
INV_SCALE = 1.0 / 16.0

NC = 2
NS = 16
NW = NC * NS
EPW = E // NW
K = 125
CH = EPW // K
RPS = NPAD // NS

R = 1024
GB = NPAD // R

_MESH = plsc.VectorSubcoreMesh(core_axis_name="c", subcore_axis_name="s")


def _spmm_sc(nb, weighted, pexps, scalar_mode):
    def body(*refs):
        i = 0
        tbls = refs[i:i + nb]; i += nb
        srcR = refs[i]; dstR = refs[i + 1]; i += 2
        wR = None
        if weighted:
            wR = refs[i]; i += 1
        zf128 = refs[i]; i += 1
        outs = refs[i:i + nb]; i += nb
        out16 = None
        if scalar_mode is not None:
            out16 = refs[i]; i += 1
        idxs, idxd, rowsA, rowsB, w_v, acc, gsA, gsB, ssA, ssB = refs[i:]

        cid = lax.axis_index("c")
        sid = lax.axis_index("s")
        wid = sid * NC + cid
        rs = sid * RPS

        pltpu.sync_copy(srcR.at[wid], idxs)
        pltpu.sync_copy(dstR.at[wid], idxd)
        if weighted:
            pltpu.sync_copy(wR.at[wid], w_v)

        def mulw(buf, cbase, p):
            def rowmul(r, _2):
                w = w_v[pl.ds(cbase * K + r, 16)][0]
                if p == 2:
                    w = w * w
                for j in range(8):
                    sl = pl.ds(j * 16, 16)
                    buf[r, sl] = buf[r, sl] * w
                return 0
            lax.fori_loop(0, K, rowmul, 0)

        def drainA(c):
            pltpu.make_async_copy(rowsA, acc.at[idxd.at[c]], ssA).wait()

        def drainB(c):
            pltpu.make_async_copy(rowsB, acc.at[idxd.at[c]], ssB).wait()

        for b in range(nb):
            pltpu.sync_copy(zf128, acc.at[pl.ds(rs, RPS)])
            plsc.subcore_barrier()
            p = pexps[b] if weighted else 1

            def pair(g, _, b=b, p=p):
                c0 = 2 * g
                c1 = c0 + 1

                @pl.when(g > 0)
                def _():
                    drainA(c0 - 2)

                pltpu.async_copy(tbls[b].at[idxs.at[c0]], rowsA,
                                 gsA).wait()

                @pl.when(g > 0)
                def _():
                    drainB(c1 - 2)

                dB = pltpu.async_copy(tbls[b].at[idxs.at[c1]], rowsB,
                                      gsB)
                if weighted:
                    mulw(rowsA, c0, p)
                pltpu.async_copy(rowsA, acc.at[idxd.at[c0]], ssA,
                                 add=True)
                dB.wait()
                if weighted:
                    mulw(rowsB, c1, p)
                pltpu.async_copy(rowsB, acc.at[idxd.at[c1]], ssB,
                                 add=True)
                return 0

            lax.fori_loop(0, CH // 2, pair, 0)
            drainA(CH - 2)
            drainB(CH - 1)
            plsc.subcore_barrier()
            pltpu.sync_copy(acc.at[pl.ds(rs, RPS)],
                            outs[b].at[cid, pl.ds(rs, RPS)])
            plsc.subcore_barrier()

        if scalar_mode is not None:
            lane0 = jnp.where(lax.iota(jnp.int32, 16) == 0,
                              jnp.float32(1.0), jnp.float32(0.0))
            zero16 = jnp.zeros((16,), jnp.float32)
            pltpu.sync_copy(zf128, acc.at[pl.ds(rs, RPS)])
            first = lane0 if scalar_mode == "ones" else zero16

            def fill0(r, _2):
                for j in range(1, 8):
                    rowsA[r, pl.ds(j * 16, 16)] = zero16
                    rowsB[r, pl.ds(j * 16, 16)] = zero16
                rowsA[r, pl.ds(0, 16)] = first
                rowsB[r, pl.ds(0, 16)] = first
                return 0

            lax.fori_loop(0, K, fill0, 0)
            plsc.subcore_barrier()

            def fillw(buf, cbase):
                def fr(r, _2):
                    w = w_v[pl.ds(cbase * K + r, 16)][0]
                    buf[r, pl.ds(0, 16)] = lane0 * w
                    return 0
                lax.fori_loop(0, K, fr, 0)

            def pair16(g, _):
                c0 = 2 * g
                c1 = c0 + 1

                @pl.when(g > 0)
                def _():
                    drainA(c0 - 2)

                if scalar_mode == "w":
                    fillw(rowsA, c0)
                pltpu.async_copy(rowsA, acc.at[idxd.at[c0]], ssA,
                                 add=True)

                @pl.when(g > 0)
                def _():
                    drainB(c1 - 2)

                if scalar_mode == "w":
                    fillw(rowsB, c1)
                pltpu.async_copy(rowsB, acc.at[idxd.at[c1]], ssB,
                                 add=True)
                return 0

            lax.fori_loop(0, CH // 2, pair16, 0)
            drainA(CH - 2)
            drainB(CH - 1)
            plsc.subcore_barrier()
            pltpu.sync_copy(acc.at[pl.ds(rs, RPS)],
                            out16.at[cid, pl.ds(rs, RPS)])

    out_type = [jax.ShapeDtypeStruct((NC, NPAD, 128), jnp.float32)
                for _ in range(nb)]
    if scalar_mode is not None:
        out_type.append(jax.ShapeDtypeStruct((NC, NPAD, 128), jnp.float32))
    scratch = [
        pltpu.VMEM((CH, K), jnp.int32),
        pltpu.VMEM((CH, K), jnp.int32),
        pltpu.VMEM((K, 128), jnp.float32),
        pltpu.VMEM((K, 128), jnp.float32),
        pltpu.VMEM((EPW + 16,), jnp.float32),
        pltpu.VMEM_SHARED((NPAD, 128), jnp.float32),
        pltpu.SemaphoreType.DMA,
        pltpu.SemaphoreType.DMA,
        pltpu.SemaphoreType.DMA,
        pltpu.SemaphoreType.DMA,
    ]
    return pl.kernel(body, out_type=tuple(out_type), mesh=_MESH,
                     scratch_types=scratch)


KD = 50
CHD = EPW // KD
KDP = 56


def _edge_dot_sc():
    def body(qt0, qt1, kt0, kt1, srcR, dstR, dots_out, idxs, idxd,
             qA0, qA1, kA0, kA1, qB0, qB1, kB0, kB1, dots_v, gsA, gsB):
        cid = lax.axis_index("c")
        sid = lax.axis_index("s")
        wid = sid * NC + cid
        pltpu.sync_copy(srcR.at[wid], idxs)
        pltpu.sync_copy(dstR.at[wid], idxd)
        zero16 = jnp.zeros((16,), jnp.float32)
        for r in range(KD, KDP):
            dots_v[r, pl.ds(0, 16)] = zero16

        def gather4(c, bufs, sem):
            pltpu.async_copy(qt0.at[idxd.at[c]], bufs[0], sem)
            pltpu.async_copy(qt1.at[idxd.at[c]], bufs[1], sem)
            pltpu.async_copy(kt0.at[idxs.at[c]], bufs[2], sem)
            pltpu.async_copy(kt1.at[idxs.at[c]], bufs[3], sem)

        def wait4(c, bufs, sem):
            pltpu.make_async_copy(qt0.at[idxd.at[c]], bufs[0], sem).wait()
            pltpu.make_async_copy(qt1.at[idxd.at[c]], bufs[1], sem).wait()
            pltpu.make_async_copy(kt0.at[idxs.at[c]], bufs[2], sem).wait()
            pltpu.make_async_copy(kt1.at[idxs.at[c]], bufs[3], sem).wait()

        def compute(c, bufs):
            q0, q1, k0, k1 = bufs

            def edge(r, _2):
                acc = q0[r, pl.ds(0, 16)] * k0[r, pl.ds(0, 16)]
                for j in range(1, 8):
                    sl = pl.ds(j * 16, 16)
                    acc = acc + q0[r, sl] * k0[r, sl]
                for j in range(8):
                    sl = pl.ds(j * 16, 16)
                    acc = acc + q1[r, sl] * k1[r, sl]
                dots_v[r, pl.ds(0, 16)] = acc
                return 0

            lax.fori_loop(0, KD, edge, 0)
            pltpu.sync_copy(dots_v, dots_out.at[wid, c])

        bufsA = (qA0, qA1, kA0, kA1)
        bufsB = (qB0, qB1, kB0, kB1)
        gather4(0, bufsA, gsA)

        def pair(g, _):
            c0 = 2 * g
            c1 = c0 + 1
            gather4(c1, bufsB, gsB)
            wait4(c0, bufsA, gsA)
            compute(c0, bufsA)

            @pl.when(g < CHD // 2 - 1)
            def _():
                gather4(c0 + 2, bufsA, gsA)

            wait4(c1, bufsB, gsB)
            compute(c1, bufsB)
            return 0

        lax.fori_loop(0, CHD // 2, pair, 0)

    out_type = jax.ShapeDtypeStruct((NW, CHD, KDP, 16), jnp.float32)
    scratch = (
        [pltpu.VMEM((CHD, KD), jnp.int32)] * 2
        + [pltpu.VMEM((KD, 128), jnp.float32)] * 8
        + [pltpu.VMEM((KDP, 16), jnp.float32),
           pltpu.SemaphoreType.DMA, pltpu.SemaphoreType.DMA]
    )
    return pl.kernel(body, out_type=out_type, mesh=_MESH,
                     scratch_types=scratch)


def _row_spec(cols):
    return pl.BlockSpec((R, cols), lambda i: (i, 0))


def _part_spec(cols):
    return pl.BlockSpec((NC, R, cols), lambda i: (0, i, 0))


def _whole(shape):
    nd = len(shape)
    return pl.BlockSpec(shape, lambda i=0: (0,) * nd)


def _std_from_parts(p0, p1, p2, p3, c16):
    s1 = jnp.concatenate([p0[0] + p0[1], p1[0] + p1[1]], axis=-1)
    s2 = jnp.concatenate([p2[0] + p2[1], p3[0] + p3[1]], axis=-1)
    cntc = jnp.maximum((c16[0] + c16[1])[:, :1], 1.0)
    mean = s1 / cntc
    m2 = s2 / cntc
    return jnp.sqrt(jnp.maximum(m2 - mean * mean, 1e-5)), cntc


def _stage_a(x, wsrc, bsrc, wdst, bdst):
    def body(x_r, ws_r, bs_r, wd_r, bd_r, t0, t1, t2, t3, xd):
        xb = x_r[...]
        m = jnp.maximum(jnp.dot(xb, ws_r[...],
                                preferred_element_type=jnp.float32)
                        + bs_r[...], 0.0) + 1e-7
        ms = m * m
        t0[...] = m[:, :128]
        t1[...] = m[:, 128:]
        t2[...] = ms[:, :128]
        t3[...] = ms[:, 128:]
        xd[...] = jnp.dot(xb, wd_r[...],
                          preferred_element_type=jnp.float32) + bd_r[...]

    outs = [jax.ShapeDtypeStruct((NPAD, 128), jnp.float32)] * 4 + [
        jax.ShapeDtypeStruct((NPAD, 256), jnp.float32)]
    return pl.pallas_call(
        body, grid=(GB,),
        in_specs=[_row_spec(256), _whole((256, 256)), _whole((1, 256)),
                  _whole((256, 256)), _whole((1, 256))],
        out_specs=[_row_spec(128)] * 4 + [_row_spec(256)],
        out_shape=outs,
    )(x, wsrc, bsrc, wdst, bdst)


def _stage_b1(p0, p1, p2, p3, c16, xdst, w1, b1):
    def body(p0r, p1r, p2r, p3r, cr, xdr, w1r, b1r, hmid, psum, psq):
        std, _ = _std_from_parts(p0r[...], p1r[...], p2r[...], p3r[...],
                                 cr[...])
        h = std + xdr[...]
        hm = jnp.dot(h, w1r[...], preferred_element_type=jnp.float32) \
            + b1r[...]
        hmid[...] = hm
        rid = pl.program_id(0) * R + lax.broadcasted_iota(
            jnp.int32, (R, 1), 0)
        hmm = hm * (rid < N).astype(jnp.float32)
        psum[...] = jnp.sum(hmm, axis=0, keepdims=True)[None]
        psq[...] = jnp.sum(hmm * hmm, axis=0, keepdims=True)[None]

    outs = [jax.ShapeDtypeStruct((NPAD, 512), jnp.float32),
            jax.ShapeDtypeStruct((GB, 1, 512), jnp.float32),
            jax.ShapeDtypeStruct((GB, 1, 512), jnp.float32)]
    stat_spec = pl.BlockSpec((1, 1, 512), lambda i: (i, 0, 0))
    return pl.pallas_call(
        body, grid=(GB,),
        in_specs=[_part_spec(128)] * 5 + [_row_spec(256),
                  _whole((256, 512)), _whole((1, 512))],
        out_specs=[_row_spec(512), stat_spec, stat_spec],
        out_shape=outs,
    )(p0, p1, p2, p3, c16, xdst, w1, b1)


def _stage_b3(hmid, psum, psq, gamma, beta, w2, b2):
    def body(hm_r, ps, pq, g, b, w2_r, b2_r, x1, t0, t1, t2, t3):
        mu = jnp.sum(ps[...], axis=0) / N
        var = jnp.sum(pq[...], axis=0) / N - mu * mu
        inv = 1.0 / jnp.sqrt(var + 1e-5)
        gs = g[...] * inv
        gb = b[...] - mu * g[...] * inv
        hb = jnp.maximum(hm_r[...] * gs + gb, 0.0)
        x1b = jnp.maximum(jnp.dot(hb, w2_r[...],
                                  preferred_element_type=jnp.float32)
                          + b2_r[...], 0.0)
        x1s = x1b * x1b
        x1[...] = x1b
        t0[...] = x1b[:, :128]
        t1[...] = x1b[:, 128:]
        t2[...] = x1s[:, :128]
        t3[...] = x1s[:, 128:]

    outs = [jax.ShapeDtypeStruct((NPAD, 256), jnp.float32)] + [
        jax.ShapeDtypeStruct((NPAD, 128), jnp.float32)] * 4
    return pl.pallas_call(
        body, grid=(GB,),
        in_specs=[_row_spec(512), _whole((GB, 1, 512)),
                  _whole((GB, 1, 512)), _whole((1, 512)),
                  _whole((1, 512)), _whole((512, 256)), _whole((1, 256))],
        out_specs=[_row_spec(256)] + [_row_spec(128)] * 4,
        out_shape=outs,
    )(hmid, psum, psq, gamma, beta, w2, b2)


def _stage_c1(p0, p1, p2, p3, c16, x1, wl, bl, wr, wq, bq, wk, bk,
              wv, bv, wsk, bsk):
    def body(p0r, p1r, p2r, p3r, cr, x1r, wlr, blr, wrr, wqr, bqr, wkr,
             bkr, wvr, bvr, wskr, bskr, qo, ko, t0, t1, t2, t3, sk):
        std, _ = _std_from_parts(p0r[...], p1r[...], p2r[...], p3r[...],
                                 cr[...])
        x1b = x1r[...]
        x2 = jnp.maximum(
            jnp.dot(std, wlr[...], preferred_element_type=jnp.float32)
            + blr[...]
            + jnp.dot(x1b, wrr[...], preferred_element_type=jnp.float32),
            0.0)
        qo[...] = jnp.dot(x2, wqr[...],
                          preferred_element_type=jnp.float32) + bqr[...]
        ko[...] = jnp.dot(x2, wkr[...],
                          preferred_element_type=jnp.float32) + bkr[...]
        v = jnp.dot(x2, wvr[...],
                    preferred_element_type=jnp.float32) + bvr[...]
        vs = v * v
        t0[...] = v[:, :128]
        t1[...] = v[:, 128:]
        t2[...] = vs[:, :128]
        t3[...] = vs[:, 128:]
        sk[...] = jnp.dot(x2, wskr[...],
                          preferred_element_type=jnp.float32) + bskr[...]

    outs = ([jax.ShapeDtypeStruct((NPAD, 256), jnp.float32)] * 2
            + [jax.ShapeDtypeStruct((NPAD, 128), jnp.float32)] * 4
            + [jax.ShapeDtypeStruct((NPAD, 256), jnp.float32)])
    wspec = _whole((256, 256))
    bspec = _whole((1, 256))
    return pl.pallas_call(
        body, grid=(GB,),
        in_specs=[_part_spec(128)] * 5 + [_row_spec(256),
                  wspec, bspec, wspec, wspec, bspec, wspec, bspec,
                  wspec, bspec, wspec, bspec],
        out_specs=[_row_spec(256)] * 2 + [_row_spec(128)] * 4
        + [_row_spec(256)],
        out_shape=outs,
    )(p0, p1, p2, p3, c16, x1, wl, bl, wr, wq, bq, wk, bk, wv, bv,
      wsk, bsk)


C2R = 2000
C2G = E // C2R


def _stage_c2a(dots16):
    def body(d_r, ds_o, pm_o):
        s = jnp.sum(d_r[...], axis=1, keepdims=True) * INV_SCALE
        ds_o[...] = s
        pm_o[...] = jnp.max(s) + jnp.zeros((1, 1, 128), jnp.float32)

    return pl.pallas_call(
        body, grid=(C2G,),
        in_specs=[pl.BlockSpec((C2R, 16), lambda i: (i, 0))],
        out_specs=[pl.BlockSpec((C2R, 1), lambda i: (i, 0)),
                   pl.BlockSpec((1, 1, 128), lambda i: (i, 0, 0))],
        out_shape=[jax.ShapeDtypeStruct((E, 1), jnp.float32),
                   jax.ShapeDtypeStruct((C2G, 1, 128), jnp.float32)],
    )(dots16)


def _stage_c2b(dots2d, pmax):
    def body(d_r, m_r, w):
        kmax = jnp.max(m_r[...])
        w[...] = jnp.exp(d_r[...] - kmax)

    return pl.pallas_call(
        body,
        in_specs=[_whole((E // 128, 128)), _whole((C2G, 1, 128))],
        out_specs=_whole((E // 128, 128)),
        out_shape=jax.ShapeDtypeStruct((E // 128, 128), jnp.float32),
    )(dots2d, pmax)


def _stage_d(p0, p1, p2, p3, d16, c16, skip):
    def body(p0r, p1r, p2r, p3r, dr, cr, skr, out):
        d = (dr[0] + dr[1])[:, :1] + 1e-16
        s1 = jnp.concatenate([p0r[0] + p0r[1], p1r[0] + p1r[1]],
                             axis=-1) / d
        s2 = jnp.concatenate([p2r[0] + p2r[1], p3r[0] + p3r[1]],
                             axis=-1) / (d * d)
        cntc = jnp.maximum((cr[0] + cr[1])[:, :1], 1.0)
        mean = s1 / cntc
        m2 = s2 / cntc
        std = jnp.sqrt(jnp.maximum(m2 - mean * mean, 1e-5))
        out[...] = std + skr[...]

    return pl.pallas_call(
        body, grid=(GB,),
        in_specs=[_part_spec(128)] * 6 + [_row_spec(256)],
        out_specs=_row_spec(256),
        out_shape=jax.ShapeDtypeStruct((NPAD, 256), jnp.float32),
    )(p0, p1, p2, p3, d16, c16, skip)


def kernel(x, edge_index, gen_lin_src_w, gen_lin_src_b, gen_lin_dst_w,
           gen_lin_dst_b, gen_mlp_w1, gen_mlp_b1, gen_bn_gamma,
           gen_bn_beta, gen_mlp_w2, gen_mlp_b2, sage_lin_l_w,
           sage_lin_l_b, sage_lin_r_w, tr_q_w, tr_q_b, tr_k_w, tr_k_b,
           tr_v_w, tr_v_b, tr_skip_w, tr_skip_b):
    xp = jnp.pad(x, ((0, NPAD - N), (0, 0)))
    srcW = edge_index[0].reshape(NW, CH, K)
    dstW = edge_index[1].reshape(NW, CH, K)
    zf128 = jnp.zeros((RPS, 128), jnp.float32)
    r2 = lambda b: b.reshape(1, -1)

    t0, t1, t2, t3, xdst = _stage_a(xp, gen_lin_src_w, r2(gen_lin_src_b),
                                    gen_lin_dst_w, r2(gen_lin_dst_b))
    p0, p1, p2, p3, c16 = _spmm_sc(4, False, None, "ones")(
        t0, t1, t2, t3, srcW, dstW, zf128)
    hmid, psum, psq = _stage_b1(p0, p1, p2, p3, c16, xdst,
                                gen_mlp_w1, r2(gen_mlp_b1))
    x1, u0, u1, u2, u3 = _stage_b3(hmid, psum, psq, r2(gen_bn_gamma),
                                   r2(gen_bn_beta), gen_mlp_w2,
                                   r2(gen_mlp_b2))
    q0, q1, q2, q3 = _spmm_sc(4, False, None, None)(
        u0, u1, u2, u3, srcW, dstW, zf128)
    qt, kt, v0, v1, v2, v3, skip = _stage_c1(
        q0, q1, q2, q3, c16, x1, sage_lin_l_w, r2(sage_lin_l_b),
        sage_lin_r_w, tr_q_w, r2(tr_q_b), tr_k_w, r2(tr_k_b), tr_v_w,
        r2(tr_v_b), tr_skip_w, r2(tr_skip_b))
    srcD = edge_index[0].reshape(NW, CHD, KD)
    dstD = edge_index[1].reshape(NW, CHD, KD)
    dots16 = _edge_dot_sc()(qt[:, :128], qt[:, 128:], kt[:, :128],
                            kt[:, 128:], srcD, dstD)[:, :, :KD, :]
    dsum, pmax = _stage_c2a(dots16.reshape(E, 16))
    w = _stage_c2b(dsum.reshape(E // 128, 128), pmax)
    wW = jnp.pad(w.reshape(NW, EPW), ((0, 0), (0, 16)))
    r0, r1, r2_, r3, d16 = _spmm_sc(4, True, (1, 1, 2, 2), "w")(
        v0, v1, v2, v3, srcW, dstW, wW, zf128)
    out = _stage_d(r0, r1, r2_, r3, d16, c16, skip)
    return out[:N]


# speedup vs baseline: 5.0117x; 1.0046x over previous
import functools

import jax
import jax.numpy as jnp
from jax import lax
from jax.experimental import pallas as pl
from jax.experimental.pallas import tpu as pltpu
from jax.experimental.pallas import tpu_sc as plsc

N = 10000
NPAD = 10240
E = 160000
H = 256
INV_SCALE = 1.0 / 16.0

NC = 2
NS = 16
NW = NC * NS
EPW = E // NW
K = 125
CH = EPW // K
RPS = NPAD // NS

R = 2048
GB = NPAD // R

_MESH = plsc.VectorSubcoreMesh(core_axis_name="c", subcore_axis_name="s")


def _spmm_sc(nb, weighted, pexps, scalar_mode):
    def body(*refs):
        i = 0
        tbls = refs[i:i + nb]; i += nb
        srcR = refs[i]; dstR = refs[i + 1]; i += 2
        wR = None
        if weighted:
            wR = refs[i]; i += 1
        zf128 = refs[i]; i += 1
        outs = refs[i:i + nb]; i += nb
        out16 = None
        if scalar_mode is not None:
            out16 = refs[i]; i += 1
        idxs, idxd, rowsA, rowsB, w_v, acc, gsA, gsB, ssA, ssB = refs[i:]

        cid = lax.axis_index("c")
        sid = lax.axis_index("s")
        wid = sid * NC + cid
        rs = sid * RPS

        pltpu.sync_copy(srcR.at[wid], idxs)
        pltpu.sync_copy(dstR.at[wid], idxd)
        if weighted:
            pltpu.sync_copy(wR.at[wid], w_v)

        def mulw(buf, cbase, p):
            def rowmul(r, _2):
                w = w_v[pl.ds(cbase * K + r, 16)][0]
                if p == 2:
                    w = w * w
                for j in range(8):
                    sl = pl.ds(j * 16, 16)
                    buf[r, sl] = buf[r, sl] * w
                return 0
            lax.fori_loop(0, K, rowmul, 0)

        def drainA(c):
            pltpu.make_async_copy(rowsA, acc.at[idxd.at[c]], ssA).wait()

        def drainB(c):
            pltpu.make_async_copy(rowsB, acc.at[idxd.at[c]], ssB).wait()

        for b in range(nb):
            pltpu.sync_copy(zf128, acc.at[pl.ds(rs, RPS)])
            plsc.subcore_barrier()
            p = pexps[b] if weighted else 1

            def pair(g, _, b=b, p=p):
                c0 = 2 * g
                c1 = c0 + 1

                @pl.when(g > 0)
                def _():
                    drainA(c0 - 2)

                pltpu.async_copy(tbls[b].at[idxs.at[c0]], rowsA,
                                 gsA).wait()

                @pl.when(g > 0)
                def _():
                    drainB(c1 - 2)

                dB = pltpu.async_copy(tbls[b].at[idxs.at[c1]], rowsB,
                                      gsB)
                if weighted:
                    mulw(rowsA, c0, p)
                pltpu.async_copy(rowsA, acc.at[idxd.at[c0]], ssA,
                                 add=True)
                dB.wait()
                if weighted:
                    mulw(rowsB, c1, p)
                pltpu.async_copy(rowsB, acc.at[idxd.at[c1]], ssB,
                                 add=True)
                return 0

            lax.fori_loop(0, CH // 2, pair, 0)
            drainA(CH - 2)
            drainB(CH - 1)
            plsc.subcore_barrier()
            pltpu.sync_copy(acc.at[pl.ds(rs, RPS)],
                            outs[b].at[cid, pl.ds(rs, RPS)])
            plsc.subcore_barrier()

        if scalar_mode is not None:
            lane0 = jnp.where(lax.iota(jnp.int32, 16) == 0,
                              jnp.float32(1.0), jnp.float32(0.0))
            zero16 = jnp.zeros((16,), jnp.float32)
            pltpu.sync_copy(zf128, acc.at[pl.ds(rs, RPS)])
            first = lane0 if scalar_mode == "ones" else zero16

            def fill0(r, _2):
                for j in range(1, 8):
                    rowsA[r, pl.ds(j * 16, 16)] = zero16
                    rowsB[r, pl.ds(j * 16, 16)] = zero16
                rowsA[r, pl.ds(0, 16)] = first
                rowsB[r, pl.ds(0, 16)] = first
                return 0

            lax.fori_loop(0, K, fill0, 0)
            plsc.subcore_barrier()

            def fillw(buf, cbase):
                def fr(r, _2):
                    w = w_v[pl.ds(cbase * K + r, 16)][0]
                    buf[r, pl.ds(0, 16)] = lane0 * w
                    return 0
                lax.fori_loop(0, K, fr, 0)

            def pair16(g, _):
                c0 = 2 * g
                c1 = c0 + 1

                @pl.when(g > 0)
                def _():
                    drainA(c0 - 2)

                if scalar_mode == "w":
                    fillw(rowsA, c0)
                pltpu.async_copy(rowsA, acc.at[idxd.at[c0]], ssA,
                                 add=True)

                @pl.when(g > 0)
                def _():
                    drainB(c1 - 2)

                if scalar_mode == "w":
                    fillw(rowsB, c1)
                pltpu.async_copy(rowsB, acc.at[idxd.at[c1]], ssB,
                                 add=True)
                return 0

            lax.fori_loop(0, CH // 2, pair16, 0)
            drainA(CH - 2)
            drainB(CH - 1)
            plsc.subcore_barrier()
            pltpu.sync_copy(acc.at[pl.ds(rs, RPS)],
                            out16.at[cid, pl.ds(rs, RPS)])

    out_type = [jax.ShapeDtypeStruct((NC, NPAD, 128), jnp.float32)
                for _ in range(nb)]
    if scalar_mode is not None:
        out_type.append(jax.ShapeDtypeStruct((NC, NPAD, 128), jnp.float32))
    scratch = [
        pltpu.VMEM((CH, K), jnp.int32),
        pltpu.VMEM((CH, K), jnp.int32),
        pltpu.VMEM((K, 128), jnp.float32),
        pltpu.VMEM((K, 128), jnp.float32),
        pltpu.VMEM((EPW + 16,), jnp.float32),
        pltpu.VMEM_SHARED((NPAD, 128), jnp.float32),
        pltpu.SemaphoreType.DMA,
        pltpu.SemaphoreType.DMA,
        pltpu.SemaphoreType.DMA,
        pltpu.SemaphoreType.DMA,
    ]
    return pl.kernel(body, out_type=tuple(out_type), mesh=_MESH,
                     scratch_types=scratch)


KD = 50
CHD = EPW // KD
KDP = 56


def _edge_dot_sc():
    def body(qt0, qt1, kt0, kt1, srcR, dstR, dots_out, idxs, idxd,
             qA0, qA1, kA0, kA1, qB0, qB1, kB0, kB1, dots_v, gsA, gsB):
        cid = lax.axis_index("c")
        sid = lax.axis_index("s")
        wid = sid * NC + cid
        pltpu.sync_copy(srcR.at[wid], idxs)
        pltpu.sync_copy(dstR.at[wid], idxd)
        zero16 = jnp.zeros((16,), jnp.float32)
        for r in range(KD, KDP):
            dots_v[r, pl.ds(0, 16)] = zero16

        def gather4(c, bufs, sem):
            pltpu.async_copy(qt0.at[idxd.at[c]], bufs[0], sem)
            pltpu.async_copy(qt1.at[idxd.at[c]], bufs[1], sem)
            pltpu.async_copy(kt0.at[idxs.at[c]], bufs[2], sem)
            pltpu.async_copy(kt1.at[idxs.at[c]], bufs[3], sem)

        def wait4(c, bufs, sem):
            pltpu.make_async_copy(qt0.at[idxd.at[c]], bufs[0], sem).wait()
            pltpu.make_async_copy(qt1.at[idxd.at[c]], bufs[1], sem).wait()
            pltpu.make_async_copy(kt0.at[idxs.at[c]], bufs[2], sem).wait()
            pltpu.make_async_copy(kt1.at[idxs.at[c]], bufs[3], sem).wait()

        def compute(c, bufs):
            q0, q1, k0, k1 = bufs

            def edge(r, _2):
                acc = q0[r, pl.ds(0, 16)] * k0[r, pl.ds(0, 16)]
                for j in range(1, 8):
                    sl = pl.ds(j * 16, 16)
                    acc = acc + q0[r, sl] * k0[r, sl]
                for j in range(8):
                    sl = pl.ds(j * 16, 16)
                    acc = acc + q1[r, sl] * k1[r, sl]
                dots_v[r, pl.ds(0, 16)] = acc
                return 0

            lax.fori_loop(0, KD, edge, 0)
            pltpu.sync_copy(dots_v, dots_out.at[wid, c])

        bufsA = (qA0, qA1, kA0, kA1)
        bufsB = (qB0, qB1, kB0, kB1)
        gather4(0, bufsA, gsA)

        def pair(g, _):
            c0 = 2 * g
            c1 = c0 + 1
            gather4(c1, bufsB, gsB)
            wait4(c0, bufsA, gsA)
            compute(c0, bufsA)

            @pl.when(g < CHD // 2 - 1)
            def _():
                gather4(c0 + 2, bufsA, gsA)

            wait4(c1, bufsB, gsB)
            compute(c1, bufsB)
            return 0

        lax.fori_loop(0, CHD // 2, pair, 0)

    out_type = jax.ShapeDtypeStruct((NW, CHD, KDP, 16), jnp.float32)
    scratch = (
        [pltpu.VMEM((CHD, KD), jnp.int32)] * 2
        + [pltpu.VMEM((KD, 128), jnp.float32)] * 8
        + [pltpu.VMEM((KDP, 16), jnp.float32),
           pltpu.SemaphoreType.DMA, pltpu.SemaphoreType.DMA]
    )
    return pl.kernel(body, out_type=out_type, mesh=_MESH,
                     scratch_types=scratch)


def _row_spec(cols):
    return pl.BlockSpec((R, cols), lambda i: (i, 0))


def _part_spec(cols):
    return pl.BlockSpec((NC, R, cols), lambda i: (0, i, 0))


def _whole(shape):
    nd = len(shape)
    return pl.BlockSpec(shape, lambda i=0: (0,) * nd)


def _std_from_parts(p0, p1, p2, p3, c16):
    s1 = jnp.concatenate([p0[0] + p0[1], p1[0] + p1[1]], axis=-1)
    s2 = jnp.concatenate([p2[0] + p2[1], p3[0] + p3[1]], axis=-1)
    cntc = jnp.maximum((c16[0] + c16[1])[:, :1], 1.0)
    mean = s1 / cntc
    m2 = s2 / cntc
    return jnp.sqrt(jnp.maximum(m2 - mean * mean, 1e-5)), cntc


def _stage_a(x, wsrc, bsrc, wdst, bdst):
    def body(x_r, ws_r, bs_r, wd_r, bd_r, t0, t1, t2, t3, xd):
        xb = x_r[...]
        m = jnp.maximum(jnp.dot(xb, ws_r[...],
                                preferred_element_type=jnp.float32)
                        + bs_r[...], 0.0) + 1e-7
        ms = m * m
        t0[...] = m[:, :128]
        t1[...] = m[:, 128:]
        t2[...] = ms[:, :128]
        t3[...] = ms[:, 128:]
        xd[...] = jnp.dot(xb, wd_r[...],
                          preferred_element_type=jnp.float32) + bd_r[...]

    outs = [jax.ShapeDtypeStruct((NPAD, 128), jnp.float32)] * 4 + [
        jax.ShapeDtypeStruct((NPAD, 256), jnp.float32)]
    return pl.pallas_call(
        body, grid=(GB,),
        in_specs=[_row_spec(256), _whole((256, 256)), _whole((1, 256)),
                  _whole((256, 256)), _whole((1, 256))],
        out_specs=[_row_spec(128)] * 4 + [_row_spec(256)],
        out_shape=outs,
    )(x, wsrc, bsrc, wdst, bdst)


def _stage_b1(p0, p1, p2, p3, c16, xdst, w1, b1):
    def body(p0r, p1r, p2r, p3r, cr, xdr, w1r, b1r, hmid, psum, psq):
        std, _ = _std_from_parts(p0r[...], p1r[...], p2r[...], p3r[...],
                                 cr[...])
        h = std + xdr[...]
        hm = jnp.dot(h, w1r[...], preferred_element_type=jnp.float32) \
            + b1r[...]
        hmid[...] = hm
        rid = pl.program_id(0) * R + lax.broadcasted_iota(
            jnp.int32, (R, 1), 0)
        hmm = hm * (rid < N).astype(jnp.float32)
        psum[...] = jnp.sum(hmm, axis=0, keepdims=True)[None]
        psq[...] = jnp.sum(hmm * hmm, axis=0, keepdims=True)[None]

    outs = [jax.ShapeDtypeStruct((NPAD, 512), jnp.float32),
            jax.ShapeDtypeStruct((GB, 1, 512), jnp.float32),
            jax.ShapeDtypeStruct((GB, 1, 512), jnp.float32)]
    stat_spec = pl.BlockSpec((1, 1, 512), lambda i: (i, 0, 0))
    return pl.pallas_call(
        body, grid=(GB,),
        in_specs=[_part_spec(128)] * 5 + [_row_spec(256),
                  _whole((256, 512)), _whole((1, 512))],
        out_specs=[_row_spec(512), stat_spec, stat_spec],
        out_shape=outs,
    )(p0, p1, p2, p3, c16, xdst, w1, b1)


def _stage_b3(hmid, psum, psq, gamma, beta, w2, b2):
    def body(hm_r, ps, pq, g, b, w2_r, b2_r, x1, t0, t1, t2, t3):
        mu = jnp.sum(ps[...], axis=0) / N
        var = jnp.sum(pq[...], axis=0) / N - mu * mu
        inv = 1.0 / jnp.sqrt(var + 1e-5)
        gs = g[...] * inv
        gb = b[...] - mu * g[...] * inv
        hb = jnp.maximum(hm_r[...] * gs + gb, 0.0)
        x1b = jnp.maximum(jnp.dot(hb, w2_r[...],
                                  preferred_element_type=jnp.float32)
                          + b2_r[...], 0.0)
        x1s = x1b * x1b
        x1[...] = x1b
        t0[...] = x1b[:, :128]
        t1[...] = x1b[:, 128:]
        t2[...] = x1s[:, :128]
        t3[...] = x1s[:, 128:]

    outs = [jax.ShapeDtypeStruct((NPAD, 256), jnp.float32)] + [
        jax.ShapeDtypeStruct((NPAD, 128), jnp.float32)] * 4
    return pl.pallas_call(
        body, grid=(GB,),
        in_specs=[_row_spec(512), _whole((GB, 1, 512)),
                  _whole((GB, 1, 512)), _whole((1, 512)),
                  _whole((1, 512)), _whole((512, 256)), _whole((1, 256))],
        out_specs=[_row_spec(256)] + [_row_spec(128)] * 4,
        out_shape=outs,
    )(hmid, psum, psq, gamma, beta, w2, b2)


def _stage_c1(p0, p1, p2, p3, c16, x1, wl, bl, wr, wq, bq, wk, bk,
              wv, bv, wsk, bsk):
    def body(p0r, p1r, p2r, p3r, cr, x1r, wlr, blr, wrr, wqr, bqr, wkr,
             bkr, wvr, bvr, wskr, bskr, qo, ko, t0, t1, t2, t3, sk):
        std, _ = _std_from_parts(p0r[...], p1r[...], p2r[...], p3r[...],
                                 cr[...])
        x1b = x1r[...]
        x2 = jnp.maximum(
            jnp.dot(std, wlr[...], preferred_element_type=jnp.float32)
            + blr[...]
            + jnp.dot(x1b, wrr[...], preferred_element_type=jnp.float32),
            0.0)
        qo[...] = jnp.dot(x2, wqr[...],
                          preferred_element_type=jnp.float32) + bqr[...]
        ko[...] = jnp.dot(x2, wkr[...],
                          preferred_element_type=jnp.float32) + bkr[...]
        v = jnp.dot(x2, wvr[...],
                    preferred_element_type=jnp.float32) + bvr[...]
        vs = v * v
        t0[...] = v[:, :128]
        t1[...] = v[:, 128:]
        t2[...] = vs[:, :128]
        t3[...] = vs[:, 128:]
        sk[...] = jnp.dot(x2, wskr[...],
                          preferred_element_type=jnp.float32) + bskr[...]

    outs = ([jax.ShapeDtypeStruct((NPAD, 256), jnp.float32)] * 2
            + [jax.ShapeDtypeStruct((NPAD, 128), jnp.float32)] * 4
            + [jax.ShapeDtypeStruct((NPAD, 256), jnp.float32)])
    wspec = _whole((256, 256))
    bspec = _whole((1, 256))
    return pl.pallas_call(
        body, grid=(GB,),
        in_specs=[_part_spec(128)] * 5 + [_row_spec(256),
                  wspec, bspec, wspec, wspec, bspec, wspec, bspec,
                  wspec, bspec, wspec, bspec],
        out_specs=[_row_spec(256)] * 2 + [_row_spec(128)] * 4
        + [_row_spec(256)],
        out_shape=outs,
    )(p0, p1, p2, p3, c16, x1, wl, bl, wr, wq, bq, wk, bk, wv, bv,
      wsk, bsk)


C2R = 2000
C2G = E // C2R


def _stage_c2a(dots16):
    def body(d_r, ds_o, pm_o):
        s = jnp.sum(d_r[...], axis=1, keepdims=True) * INV_SCALE
        ds_o[...] = s
        pm_o[...] = jnp.max(s) + jnp.zeros((1, 1, 128), jnp.float32)

    return pl.pallas_call(
        body, grid=(C2G,),
        in_specs=[pl.BlockSpec((C2R, 16), lambda i: (i, 0))],
        out_specs=[pl.BlockSpec((C2R, 1), lambda i: (i, 0)),
                   pl.BlockSpec((1, 1, 128), lambda i: (i, 0, 0))],
        out_shape=[jax.ShapeDtypeStruct((E, 1), jnp.float32),
                   jax.ShapeDtypeStruct((C2G, 1, 128), jnp.float32)],
    )(dots16)


def _stage_c2b(dots2d, pmax):
    def body(d_r, m_r, w):
        kmax = jnp.max(m_r[...])
        w[...] = jnp.exp(d_r[...] - kmax)

    return pl.pallas_call(
        body,
        in_specs=[_whole((E // 128, 128)), _whole((C2G, 1, 128))],
        out_specs=_whole((E // 128, 128)),
        out_shape=jax.ShapeDtypeStruct((E // 128, 128), jnp.float32),
    )(dots2d, pmax)


def _stage_d(p0, p1, p2, p3, d16, c16, skip):
    def body(p0r, p1r, p2r, p3r, dr, cr, skr, out):
        d = (dr[0] + dr[1])[:, :1] + 1e-16
        s1 = jnp.concatenate([p0r[0] + p0r[1], p1r[0] + p1r[1]],
                             axis=-1) / d
        s2 = jnp.concatenate([p2r[0] + p2r[1], p3r[0] + p3r[1]],
                             axis=-1) / (d * d)
        cntc = jnp.maximum((cr[0] + cr[1])[:, :1], 1.0)
        mean = s1 / cntc
        m2 = s2 / cntc
        std = jnp.sqrt(jnp.maximum(m2 - mean * mean, 1e-5))
        out[...] = std + skr[...]

    return pl.pallas_call(
        body, grid=(GB,),
        in_specs=[_part_spec(128)] * 6 + [_row_spec(256)],
        out_specs=_row_spec(256),
        out_shape=jax.ShapeDtypeStruct((NPAD, 256), jnp.float32),
    )(p0, p1, p2, p3, d16, c16, skip)


def kernel(x, edge_index, gen_lin_src_w, gen_lin_src_b, gen_lin_dst_w,
           gen_lin_dst_b, gen_mlp_w1, gen_mlp_b1, gen_bn_gamma,
           gen_bn_beta, gen_mlp_w2, gen_mlp_b2, sage_lin_l_w,
           sage_lin_l_b, sage_lin_r_w, tr_q_w, tr_q_b, tr_k_w, tr_k_b,
           tr_v_w, tr_v_b, tr_skip_w, tr_skip_b):
    xp = jnp.pad(x, ((0, NPAD - N), (0, 0)))
    srcW = edge_index[0].reshape(NW, CH, K)
    dstW = edge_index[1].reshape(NW, CH, K)
    zf128 = jnp.zeros((RPS, 128), jnp.float32)
    r2 = lambda b: b.reshape(1, -1)

    t0, t1, t2, t3, xdst = _stage_a(xp, gen_lin_src_w, r2(gen_lin_src_b),
                                    gen_lin_dst_w, r2(gen_lin_dst_b))
    p0, p1, p2, p3, c16 = _spmm_sc(4, False, None, "ones")(
        t0, t1, t2, t3, srcW, dstW, zf128)
    hmid, psum, psq = _stage_b1(p0, p1, p2, p3, c16, xdst,
                                gen_mlp_w1, r2(gen_mlp_b1))
    x1, u0, u1, u2, u3 = _stage_b3(hmid, psum, psq, r2(gen_bn_gamma),
                                   r2(gen_bn_beta), gen_mlp_w2,
                                   r2(gen_mlp_b2))
    q0, q1, q2, q3 = _spmm_sc(4, False, None, None)(
        u0, u1, u2, u3, srcW, dstW, zf128)
    qt, kt, v0, v1, v2, v3, skip = _stage_c1(
        q0, q1, q2, q3, c16, x1, sage_lin_l_w, r2(sage_lin_l_b),
        sage_lin_r_w, tr_q_w, r2(tr_q_b), tr_k_w, r2(tr_k_b), tr_v_w,
        r2(tr_v_b), tr_skip_w, r2(tr_skip_b))
    srcD = edge_index[0].reshape(NW, CHD, KD)
    dstD = edge_index[1].reshape(NW, CHD, KD)
    dots16 = _edge_dot_sc()(qt[:, :128], qt[:, 128:], kt[:, :128],
                            kt[:, 128:], srcD, dstD)[:, :, :KD, :]
    dsum, pmax = _stage_c2a(dots16.reshape(E, 16))
    w = _stage_c2b(dsum.reshape(E // 128, 128), pmax)
    wW = jnp.pad(w.reshape(NW, EPW), ((0, 0), (0, 16)))
    r0, r1, r2_, r3, d16 = _spmm_sc(4, True, (1, 1, 2, 2), "w")(
        v0, v1, v2, v3, srcW, dstW, wW, zf128)
    out = _stage_d(r0, r1, r2_, r3, d16, c16, skip)
    return out[:N]
